# baseline (speedup 1.0000x reference)
import functools
import jax
import jax.numpy as jnp
from jax import lax
from jax.experimental import pallas as pl
from jax.experimental.pallas import tpu as pltpu, tpu_sc as plsc

_B, _S, _E = 4, 2048, 1024
_NW = 32
_SEQ_PER_W = _S // _NW
_CH_ROWS = 4
_NCHUNK = _SEQ_PER_W // _CH_ROWS
_NBUF = 3
_VEC = 16
_CHE = _CH_ROWS * _E


def _make_sc_kernel():
    mesh = plsc.VectorSubcoreMesh(core_axis_name="c", subcore_axis_name="s")

    @functools.partial(
        pl.kernel,
        mesh=mesh,
        out_type=jax.ShapeDtypeStruct((_B, _S, _E), jnp.float32),
        scratch_types=[pltpu.VMEM((_B, _CH_ROWS, _E), jnp.float32)] * _NBUF
        + [pltpu.VMEM((_SEQ_PER_W, _E), jnp.float32)]
        + [pltpu.SemaphoreType.DMA] * (2 * _NBUF + 4),
    )
    def k(x_hbm, pos_hbm, out_hbm, *rest):
        xbufs = rest[:_NBUF]
        pbuf = rest[_NBUF]
        sems = rest[_NBUF + 1:]
        lsems, ssems = sems[:_NBUF], sems[_NBUF:2 * _NBUF]
        psems = sems[2 * _NBUF:]
        wid = lax.axis_index("s") * 2 + lax.axis_index("c")
        r0 = wid * _SEQ_PER_W

        _PP = _SEQ_PER_W // 4
        pos_loads = [
            pltpu.async_copy(
                pos_hbm.at[pl.ds(r0 + j * _PP, _PP), :],
                pbuf.at[pl.ds(j * _PP, _PP), :], psems[j],
            )
            for j in range(4)
        ]

        def load(ci):
            slot = ci % _NBUF
            r = r0 + ci * _CH_ROWS
            return pltpu.async_copy(
                x_hbm.at[:, pl.ds(r, _CH_ROWS), :], xbufs[slot], lsems[slot]
            )

        loads, stores = {}, {}
        loads[0] = load(0)

        for ci in range(_NCHUNK):
            slot = ci % _NBUF
            if ci >= 2:
                stores[ci - 2].wait()
            if ci + 1 < _NCHUNK:
                loads[ci + 1] = load(ci + 1)
            if ci % (_NCHUNK // 4) == 0:
                pos_loads[ci // (_NCHUNK // 4)].wait()
            loads[ci].wait()
            xb = xbufs[slot]
            prow0 = ci * _CH_ROWS

            @plsc.parallel_loop(0, _CHE, step=_VEC, unroll=4)
            def _(i):
                rr = lax.shift_right_logical(i, 10)
                c = pl.multiple_of(lax.bitwise_and(i, _E - 1), _VEC)
                sl = pl.ds(c, _VEC)
                pv = pbuf[prow0 + rr, sl]
                for b in range(_B):
                    xb[b, rr, sl] = xb[b, rr, sl] + pv

            r = r0 + ci * _CH_ROWS
            stores[ci] = pltpu.async_copy(
                xb, out_hbm.at[:, pl.ds(r, _CH_ROWS), :], ssems[slot]
            )
        for ci in (_NCHUNK - 2, _NCHUNK - 1):
            stores[ci].wait()

    return k


_sc_kernel = _make_sc_kernel()


def kernel(x, pos_table):
    return _sc_kernel(x, pos_table)


# speedup vs baseline: 1.0064x; 1.0064x over previous
import functools
import jax
import jax.numpy as jnp
from jax import lax
from jax.experimental import pallas as pl
from jax.experimental.pallas import tpu as pltpu, tpu_sc as plsc

_B, _S, _E = 4, 2048, 1024
_NW = 32
_SEQ_PER_W = _S // _NW
_CH_ROWS = 4
_NCHUNK = _SEQ_PER_W // _CH_ROWS
_NBUF = 3
_VEC = 16
_CHE = _CH_ROWS * _E


def _make_sc_kernel():
    mesh = plsc.VectorSubcoreMesh(core_axis_name="c", subcore_axis_name="s")

    @functools.partial(
        pl.kernel,
        mesh=mesh,
        out_type=jax.ShapeDtypeStruct((_B, _S, _E), jnp.float32),
        scratch_types=[pltpu.VMEM((_B, _CH_ROWS, _E), jnp.float32)] * _NBUF
        + [pltpu.VMEM((_SEQ_PER_W, _E), jnp.float32)]
        + [pltpu.SemaphoreType.DMA] * (2 * _NBUF + 1),
    )
    def k(x_hbm, pos_hbm, out_hbm, *rest):
        xbufs = rest[:_NBUF]
        pbuf = rest[_NBUF]
        sems = rest[_NBUF + 1:]
        lsems, ssems, psem = sems[:_NBUF], sems[_NBUF:2 * _NBUF], sems[-1]
        wid = lax.axis_index("s") * 2 + lax.axis_index("c")
        r0 = wid * _SEQ_PER_W

        pos_load = pltpu.async_copy(
            pos_hbm.at[pl.ds(r0, _SEQ_PER_W), :], pbuf, psem
        )

        def load(ci):
            slot = ci % _NBUF
            r = r0 + ci * _CH_ROWS
            return pltpu.async_copy(
                x_hbm.at[:, pl.ds(r, _CH_ROWS), :], xbufs[slot], lsems[slot]
            )

        loads, stores = {}, {}
        loads[0] = load(0)
        pos_load.wait()

        for ci in range(_NCHUNK):
            slot = ci % _NBUF
            if ci >= 2:
                stores[ci - 2].wait()
            if ci + 1 < _NCHUNK:
                loads[ci + 1] = load(ci + 1)
            loads[ci].wait()
            xb = xbufs[slot]
            prow0 = ci * _CH_ROWS

            @plsc.parallel_loop(0, _CHE, step=_VEC, unroll=4)
            def _(i):
                rr = lax.shift_right_logical(i, 10)
                c = pl.multiple_of(lax.bitwise_and(i, _E - 1), _VEC)
                sl = pl.ds(c, _VEC)
                pv = pbuf[prow0 + rr, sl]
                for b in range(_B):
                    xb[b, rr, sl] = xb[b, rr, sl] + pv

            r = r0 + ci * _CH_ROWS
            stores[ci] = pltpu.async_copy(
                xb, out_hbm.at[:, pl.ds(r, _CH_ROWS), :], ssems[slot]
            )
        for ci in (_NCHUNK - 2, _NCHUNK - 1):
            stores[ci].wait()

    return k


_sc_kernel = _make_sc_kernel()


def kernel(x, pos_table):
    return _sc_kernel(x, pos_table)
